# baseline (speedup 1.0000x reference)
import functools

import jax
import jax.numpy as jnp
from jax import lax
from jax.experimental import pallas as pl
from jax.experimental.pallas import tpu as pltpu
from jax.experimental.pallas import tpu_sc as plsc

D = 64
NNBR = 5
NC = 2
NS = 16
NW = NC * NS
GRP = 128
CHUNK_GRPS = 4


def _sc_gather(table, idx3):
    nw, ngrp, grp = idx3.shape
    rows_per_w = ngrp * grp
    nchunk = ngrp // CHUNK_GRPS
    chunk_rows = CHUNK_GRPS * grp
    mesh = plsc.VectorSubcoreMesh(core_axis_name="c", subcore_axis_name="s")

    @functools.partial(
        pl.kernel,
        out_type=jax.ShapeDtypeStruct((nw * rows_per_w, D), jnp.float32),
        mesh=mesh,
        scratch_types=[
            pltpu.VMEM((ngrp, grp), jnp.int32),
            pltpu.VMEM((chunk_rows, D), jnp.float32),
            pltpu.SemaphoreType.DMA,
        ],
        compiler_params=pltpu.CompilerParams(use_tc_tiling_on_sc=False),
    )
    def k(table_hbm, idx_hbm, out_hbm, idx_v, rows_v, sem):
        wid = lax.axis_index("s") * NC + lax.axis_index("c")
        pltpu.sync_copy(idx_hbm.at[wid], idx_v)
        base = wid * rows_per_w
        for c in range(nchunk):
            copies = [
                pltpu.async_copy(
                    table_hbm.at[idx_v.at[c * CHUNK_GRPS + j]],
                    rows_v.at[pl.ds(j * grp, grp)],
                    sem,
                )
                for j in range(CHUNK_GRPS)
            ]
            for cp in copies:
                cp.wait()
            pltpu.sync_copy(
                rows_v, out_hbm.at[pl.ds(base + c * chunk_rows, chunk_rows)]
            )

    return k(table, idx3)


def _tc_posadd(gathered, coords, Wc, bc):
    rows = gathered.shape[0]
    blk = 8192

    def body(g_ref, c_ref, wc_ref, bc_ref, o_ref):
        cc = c_ref[...]
        wc = wc_ref[...]
        o_ref[...] = (
            g_ref[...]
            + cc[:, 0:1] * wc[0:1, :]
            + cc[:, 1:2] * wc[1:2, :]
            + bc_ref[...]
        )

    return pl.pallas_call(
        body,
        grid=(rows // blk,),
        in_specs=[
            pl.BlockSpec((blk, D), lambda i: (i, 0)),
            pl.BlockSpec((blk, 2), lambda i: (i, 0)),
            pl.BlockSpec((2, D), lambda i: (0, 0)),
            pl.BlockSpec((1, D), lambda i: (0, 0)),
        ],
        out_specs=pl.BlockSpec((blk, D), lambda i: (i, 0)),
        out_shape=jax.ShapeDtypeStruct((rows, D), jnp.float32),
    )(gathered, coords, Wc, bc.reshape(1, D))


def kernel(x, table, Wc, bc):
    b = x.shape[0]
    rows = b * NNBR
    ngrp = rows // (NW * GRP)
    xr = x.reshape(b, 6, 3)
    idx3 = xr[:, 1:, 0].astype(jnp.int32).reshape(NW, ngrp, GRP)
    coords = xr[:, 1:, 1:].reshape(rows, 2)
    gathered = _sc_gather(table, idx3)
    out = _tc_posadd(gathered, coords, Wc, bc)
    return out.reshape(b, NNBR, D)
